# baseline (speedup 1.0000x reference)
import functools

import jax
import jax.numpy as jnp
from jax import lax
from jax.experimental import pallas as pl
from jax.experimental.pallas import tpu as pltpu
from jax.experimental.pallas import tpu_sc as plsc

N = 10000
E = 160000
NODE_IN = 128
EDGE_IN = 16
H = 16
STEPS = 3
S2S_ITERS = 3
READOUT = 1024

NC = 2
NS = 16
NW = NC * NS
EC = 125
EH = E
CHW = EH // (NW * EC)
ROWS = EH // EC
NPAD = 10240
STRIPE = NPAD // NS


def _proj_body(x_ref, wp_ref, bp_ref, h_ref):
    h = jnp.dot(x_ref[...], wp_ref[...], preferred_element_type=jnp.float32)
    h_ref[...] = jax.nn.relu(h + bp_ref[...])


def _msg_body(hs_ref, ea_ref, r_ref, t_ref, wm_ref, b3_ref, msg_ref):
    hs = hs_ref[...]
    ea = ea_ref[...]
    hs_b = hs.astype(jnp.bfloat16)
    ea_b = ea.astype(jnp.bfloat16)
    o = jnp.dot(ea_b, r_ref[...], preferred_element_type=jnp.float32) * \
        jnp.dot(hs_b, t_ref[...], preferred_element_type=jnp.float32)
    msg_ref[...] = (
        jnp.dot(o.astype(jnp.bfloat16), wm_ref[...],
                preferred_element_type=jnp.float32)
        + jnp.dot(hs, b3_ref[...], preferred_element_type=jnp.float32)
    )


def _gru_math(neigh, hidden, wih_t, whh_t, bih, bhh, bnn):
    m = jax.nn.relu(neigh + bnn)
    gi = jnp.dot(m, wih_t, preferred_element_type=jnp.float32) + bih
    gh = jnp.dot(hidden, whh_t, preferred_element_type=jnp.float32) + bhh
    r = jax.nn.sigmoid(gi[:, 0:H] + gh[:, 0:H])
    z = jax.nn.sigmoid(gi[:, H:2 * H] + gh[:, H:2 * H])
    n = jnp.tanh(gi[:, 2 * H:3 * H] + r * gh[:, 2 * H:3 * H])
    return (1.0 - z) * n + z * hidden


NPR = N // 8


def _gru_groups(np_ref, hid_ref, wih_ref, whh_ref, bih_ref, bhh_ref, bnn_ref):
    outs = []
    for g in range(8):
        sl = slice(g * H, (g + 1) * H)
        neigh = np_ref[0, 0:NPR, sl] + np_ref[1, 0:NPR, sl]
        outs.append(_gru_math(neigh, hid_ref[:, sl], wih_ref[...],
                              whh_ref[...], bih_ref[...], bhh_ref[...],
                              bnn_ref[...]))
    return outs


def _gru_body(np_ref, hid_ref, wih_ref, whh_ref, bih_ref, bhh_ref,
              bnn_ref, out_ref):
    out_ref[...] = jnp.concatenate(
        _gru_groups(np_ref, hid_ref, wih_ref, whh_ref, bih_ref, bhh_ref,
                    bnn_ref), axis=1)


def _tail_body(np_ref, hid_ref, wih_ref, whh_ref, bih_ref, bhh_ref,
               bnn_ref, h0_ref, lwih_ref, lwhh_ref, lbih_ref, lbhh_ref,
               wsp_ref, bsp_ref, a_ref, out_ref):
    hg = _gru_groups(np_ref, hid_ref, wih_ref, whh_ref, bih_ref, bhh_ref,
                     bnn_ref)
    feat = [jnp.concatenate([h0_ref[:, g * H:(g + 1) * H], hg[g]], axis=1)
            for g in range(8)]
    q_star = jnp.zeros((1, 4 * H), jnp.float32)
    lh = jnp.zeros((1, 2 * H), jnp.float32)
    lc = jnp.zeros((1, 2 * H), jnp.float32)
    for _ in range(S2S_ITERS):
        gates = (jnp.dot(q_star, lwih_ref[...], preferred_element_type=jnp.float32)
                 + lbih_ref[...]
                 + jnp.dot(lh, lwhh_ref[...], preferred_element_type=jnp.float32)
                 + lbhh_ref[...])
        gi = jax.nn.sigmoid(gates[:, 0:2 * H])
        gf = jax.nn.sigmoid(gates[:, 2 * H:4 * H])
        gg = jnp.tanh(gates[:, 4 * H:6 * H])
        go = jax.nn.sigmoid(gates[:, 6 * H:8 * H])
        lc = gf * lc + gi * gg
        lh = go * jnp.tanh(lc)
        e = jnp.concatenate(
            [jnp.sum(f * lh, axis=1, keepdims=True) for f in feat],
            axis=1)
        mx = jnp.max(jnp.max(e, axis=1, keepdims=True), axis=0, keepdims=True)
        ex = jnp.exp(e - mx)
        ssum = jnp.sum(jnp.sum(ex, axis=1, keepdims=True), axis=0,
                       keepdims=True)
        alpha = ex / ssum
        readout = sum(
            jnp.sum(feat[g] * alpha[:, g:g + 1], axis=0, keepdims=True)
            for g in range(8))
        q_star = jnp.concatenate([lh, readout], axis=1)
    out = jnp.dot(q_star, wsp_ref[...], preferred_element_type=jnp.float32) \
        + bsp_ref[...]
    a = a_ref[0, 0]
    out_ref[...] = jnp.where(out >= 0.0, out, a * out)


def _sc_gather_body(h_hbm, src_hbm, out_hbm, idx_v, rows_v, sem):
    c = lax.axis_index("c")
    s = lax.axis_index("s")
    base = (c * NS + s) * CHW
    pltpu.sync_copy(src_hbm.at[pl.ds(base, CHW)], idx_v)

    def fire(j, carry):
        pltpu.async_copy(h_hbm.at[idx_v.at[j]], rows_v.at[j], sem)
        return carry

    lax.fori_loop(0, CHW, fire, 0)
    pltpu.make_async_copy(out_hbm.at[pl.ds(base, CHW)], rows_v, sem).wait()
    pltpu.sync_copy(rows_v, out_hbm.at[pl.ds(base, CHW)])


def _sc_scatter_body(msg_hbm, dst_hbm, zeros_hbm, out_hbm, idx_v, rows_v,
                     accum, sem):
    c = lax.axis_index("c")
    s = lax.axis_index("s")
    base = (c * NS + s) * CHW
    pltpu.sync_copy(dst_hbm.at[pl.ds(base, CHW)], idx_v)
    pltpu.sync_copy(msg_hbm.at[pl.ds(base, CHW)], rows_v)

    @pl.when(s == 0)
    def _():
        pltpu.sync_copy(zeros_hbm, accum)

    plsc.subcore_barrier()

    def add_chunk(j, carry):
        pltpu.async_copy(rows_v.at[j], accum.at[idx_v.at[j]], sem, add=True)
        return carry

    lax.fori_loop(0, CHW, add_chunk, 0)
    pltpu.make_async_copy(msg_hbm.at[pl.ds(base, CHW)], rows_v, sem).wait()
    plsc.subcore_barrier()
    pltpu.sync_copy(accum.at[pl.ds(s * STRIPE, STRIPE)],
                    out_hbm.at[c, pl.ds(s * STRIPE, STRIPE)])


@functools.cache
def _build_sc_kernels():
    mesh = plsc.VectorSubcoreMesh(core_axis_name="c", subcore_axis_name="s",
                                  num_cores=NC, num_subcores=NS)
    gather = pl.kernel(
        _sc_gather_body,
        out_type=jax.ShapeDtypeStruct((ROWS, EC, H), jnp.float32),
        mesh=mesh,
        scratch_types=[
            pltpu.VMEM((CHW, EC), jnp.int32),
            pltpu.VMEM((CHW, EC, H), jnp.float32),
            pltpu.SemaphoreType.DMA,
        ],
        compiler_params=pltpu.CompilerParams(use_tc_tiling_on_sc=False),
    )
    scatter = pl.kernel(
        _sc_scatter_body,
        out_type=jax.ShapeDtypeStruct((NC, NPAD, H), jnp.float32),
        mesh=mesh,
        scratch_types=[
            pltpu.VMEM((CHW, EC), jnp.int32),
            pltpu.VMEM((CHW, EC, H), jnp.float32),
            pltpu.VMEM_SHARED((NPAD, H), jnp.float32),
            pltpu.SemaphoreType.DMA,
        ],
        compiler_params=pltpu.CompilerParams(use_tc_tiling_on_sc=False),
    )
    return gather, scatter


def kernel(x, edge_index, edge_attr, W_proj, b_proj, W_edge, b_edge, b_nn,
           gru_W_ih, gru_W_hh, gru_b_ih, gru_b_hh,
           lstm_W_ih, lstm_W_hh, lstm_b_ih, lstm_b_hh,
           W_sp, b_sp, prelu_a):
    f32 = jnp.float32
    src3 = edge_index[0].reshape(ROWS, EC)
    dst3 = edge_index[1].reshape(ROWS, EC)

    eye = jnp.eye(H, dtype=f32)
    eye8 = jnp.eye(8, dtype=f32)
    r_mat = jnp.kron(eye8, jnp.kron(eye, jnp.ones((1, H), f32))).astype(jnp.bfloat16)
    t_mat = jnp.kron(eye8, jnp.kron(jnp.ones((1, H), f32), eye)).astype(jnp.bfloat16)
    w_mat = jnp.kron(eye8, W_edge.reshape(EDGE_IN * H, H)).astype(jnp.bfloat16)
    b3 = jnp.kron(eye8, b_edge.reshape(H, H))
    zeros_pad = jnp.zeros((NPAD, H), f32)

    bp = b_proj.reshape(1, H)
    bnn = b_nn.reshape(1, H)
    wih_t = gru_W_ih.T
    whh_t = gru_W_hh.T
    bih = gru_b_ih.reshape(1, 3 * H)
    bhh = gru_b_hh.reshape(1, 3 * H)
    lwih_t = lstm_W_ih.T
    lwhh_t = lstm_W_hh.T
    lbih = lstm_b_ih.reshape(1, 8 * H)
    lbhh = lstm_b_hh.reshape(1, 8 * H)
    bsp = b_sp.reshape(1, READOUT)
    a2 = prelu_a.reshape(1, 1)

    h0 = pl.pallas_call(
        _proj_body,
        out_shape=jax.ShapeDtypeStruct((N, H), f32),
    )(x, W_proj, bp)

    EP = EH // 8
    BLK = 1000
    msg_call = pl.pallas_call(
        _msg_body,
        grid=(EP // BLK,),
        in_specs=[
            pl.BlockSpec((BLK, 128), lambda i: (i, 0)),
            pl.BlockSpec((BLK, 128), lambda i: (i, 0)),
            pl.BlockSpec((128, 2048), lambda i: (0, 0)),
            pl.BlockSpec((128, 2048), lambda i: (0, 0)),
            pl.BlockSpec((2048, 128), lambda i: (0, 0)),
            pl.BlockSpec((128, 128), lambda i: (0, 0)),
        ],
        out_specs=pl.BlockSpec((BLK, 128), lambda i: (i, 0)),
        out_shape=jax.ShapeDtypeStruct((EP, 128), f32),
    )
    ea_p = edge_attr.reshape(EP, 128)

    gru_call = pl.pallas_call(
        _gru_body,
        out_shape=jax.ShapeDtypeStruct((NPR, 128), f32),
    )

    sc_gather, sc_scatter = _build_sc_kernels()
    h0_p = h0.reshape(NPR, 128)
    hidden_p = h0_p
    for step in range(STEPS):
        hs3 = sc_gather(hidden_p.reshape(N, H), src3)
        hs_p = hs3.reshape(EP, 128)
        msg_p = msg_call(hs_p, ea_p, r_mat, t_mat, w_mat, b3)
        msg3 = msg_p.reshape(ROWS, EC, H)
        neigh_parts = sc_scatter(msg3, dst3, zeros_pad)
        np_p = neigh_parts.reshape(NC, NPAD // 8, 128)
        if step < STEPS - 1:
            hidden_p = gru_call(np_p, hidden_p, wih_t, whh_t,
                                bih, bhh, bnn)
        else:
            out = pl.pallas_call(
                _tail_body,
                out_shape=jax.ShapeDtypeStruct((1, READOUT), f32),
            )(np_p, hidden_p, wih_t, whh_t, bih, bhh, bnn,
              h0_p, lwih_t, lwhh_t, lbih, lbhh, W_sp, bsp, a2)
    return out


# speedup vs baseline: 1.0477x; 1.0477x over previous
import functools

import jax
import jax.numpy as jnp
from jax import lax
from jax.experimental import pallas as pl
from jax.experimental.pallas import tpu as pltpu
from jax.experimental.pallas import tpu_sc as plsc

N = 10000
E = 160000
NODE_IN = 128
EDGE_IN = 16
H = 16
STEPS = 3
S2S_ITERS = 3
READOUT = 1024

NC = 2
NS = 16
NW = NC * NS
EC = 125
EH = E
CHW = EH // (NW * EC)
ROWS = EH // EC
NPAD = 10240
STRIPE = NPAD // NS


def _proj_body(x_ref, wp_ref, bp_ref, h_ref):
    h = jnp.dot(x_ref[...], wp_ref[...], preferred_element_type=jnp.float32)
    h_ref[...] = jax.nn.relu(h + bp_ref[...])


def _msg_body(hs_ref, ea_ref, r_ref, t_ref, wm_ref, b3_ref, msg_ref):
    hs = hs_ref[...]
    ea = ea_ref[...]
    hs_b = hs.astype(jnp.bfloat16)
    ea_b = ea.astype(jnp.bfloat16)
    o = jnp.dot(ea_b, r_ref[...], preferred_element_type=jnp.float32) * \
        jnp.dot(hs_b, t_ref[...], preferred_element_type=jnp.float32)
    msg_ref[...] = (
        jnp.dot(o.astype(jnp.bfloat16), wm_ref[...],
                preferred_element_type=jnp.float32)
        + jnp.dot(hs, b3_ref[...], preferred_element_type=jnp.float32)
    )


def _gru_math(neigh, hidden, wih_t, whh_t, bih, bhh, bnn):
    m = jax.nn.relu(neigh + bnn)
    gi = jnp.dot(m, wih_t, preferred_element_type=jnp.float32) + bih
    gh = jnp.dot(hidden, whh_t, preferred_element_type=jnp.float32) + bhh
    r = jax.nn.sigmoid(gi[:, 0:H] + gh[:, 0:H])
    z = jax.nn.sigmoid(gi[:, H:2 * H] + gh[:, H:2 * H])
    n = jnp.tanh(gi[:, 2 * H:3 * H] + r * gh[:, 2 * H:3 * H])
    return (1.0 - z) * n + z * hidden


NPR = N // 8


def _gru_groups(np_ref, hid_ref, wih_ref, whh_ref, bih_ref, bhh_ref, bnn_ref):
    outs = []
    for g in range(8):
        sl = slice(g * H, (g + 1) * H)
        neigh = np_ref[0, 0:NPR, sl] + np_ref[1, 0:NPR, sl]
        outs.append(_gru_math(neigh, hid_ref[:, sl], wih_ref[...],
                              whh_ref[...], bih_ref[...], bhh_ref[...],
                              bnn_ref[...]))
    return outs


def _gru_body(np_ref, hid_ref, wih_ref, whh_ref, bih_ref, bhh_ref,
              bnn_ref, out_ref):
    out_ref[...] = jnp.concatenate(
        _gru_groups(np_ref, hid_ref, wih_ref, whh_ref, bih_ref, bhh_ref,
                    bnn_ref), axis=1)


def _tail_body(np_ref, hid_ref, wih_ref, whh_ref, bih_ref, bhh_ref,
               bnn_ref, h0_ref, lwih_ref, lwhh_ref, lbih_ref, lbhh_ref,
               wsp_ref, bsp_ref, a_ref, out_ref):
    hg = _gru_groups(np_ref, hid_ref, wih_ref, whh_ref, bih_ref, bhh_ref,
                     bnn_ref)
    feat = [jnp.concatenate([h0_ref[:, g * H:(g + 1) * H], hg[g]], axis=1)
            for g in range(8)]
    q_star = jnp.zeros((1, 4 * H), jnp.float32)
    lh = jnp.zeros((1, 2 * H), jnp.float32)
    lc = jnp.zeros((1, 2 * H), jnp.float32)
    for _ in range(S2S_ITERS):
        gates = (jnp.dot(q_star, lwih_ref[...], preferred_element_type=jnp.float32)
                 + lbih_ref[...]
                 + jnp.dot(lh, lwhh_ref[...], preferred_element_type=jnp.float32)
                 + lbhh_ref[...])
        gi = jax.nn.sigmoid(gates[:, 0:2 * H])
        gf = jax.nn.sigmoid(gates[:, 2 * H:4 * H])
        gg = jnp.tanh(gates[:, 4 * H:6 * H])
        go = jax.nn.sigmoid(gates[:, 6 * H:8 * H])
        lc = gf * lc + gi * gg
        lh = go * jnp.tanh(lc)
        e = jnp.concatenate(
            [jnp.sum(f * lh, axis=1, keepdims=True) for f in feat],
            axis=1)
        mx = jnp.max(jnp.max(e, axis=1, keepdims=True), axis=0, keepdims=True)
        ex = jnp.exp(e - mx)
        ssum = jnp.sum(jnp.sum(ex, axis=1, keepdims=True), axis=0,
                       keepdims=True)
        alpha = ex / ssum
        readout = sum(
            jnp.sum(feat[g] * alpha[:, g:g + 1], axis=0, keepdims=True)
            for g in range(8))
        q_star = jnp.concatenate([lh, readout], axis=1)
    out = jnp.dot(q_star, wsp_ref[...], preferred_element_type=jnp.float32) \
        + bsp_ref[...]
    a = a_ref[0, 0]
    out_ref[...] = jnp.where(out >= 0.0, out, a * out)


def _sc_gather_body(h_hbm, src_hbm, out_hbm, idx_v, rows_v, sem):
    c = lax.axis_index("c")
    s = lax.axis_index("s")
    base = (c * NS + s) * CHW
    pltpu.sync_copy(src_hbm.at[pl.ds(base, CHW)], idx_v)

    def fire(j, carry):
        pltpu.async_copy(h_hbm.at[idx_v.at[j]], rows_v.at[j], sem)
        return carry

    lax.fori_loop(0, CHW, fire, 0)
    pltpu.make_async_copy(out_hbm.at[pl.ds(base, CHW)], rows_v, sem).wait()
    pltpu.sync_copy(rows_v, out_hbm.at[pl.ds(base, CHW)])


def _sc_scatter_body(msg_hbm, dst_hbm, zeros_hbm, out_hbm, idx_v, rows_v,
                     accum, sem):
    c = lax.axis_index("c")
    s = lax.axis_index("s")
    base = (c * NS + s) * CHW
    pltpu.sync_copy(dst_hbm.at[pl.ds(base, CHW)], idx_v)
    pltpu.sync_copy(msg_hbm.at[pl.ds(base, CHW)], rows_v)

    @pl.when(s == 0)
    def _():
        pltpu.sync_copy(zeros_hbm, accum)

    plsc.subcore_barrier()

    def add_chunk(j, carry):
        pltpu.async_copy(rows_v.at[j], accum.at[idx_v.at[j]], sem, add=True)
        return carry

    lax.fori_loop(0, CHW, add_chunk, 0)
    pltpu.make_async_copy(msg_hbm.at[pl.ds(base, CHW)], rows_v, sem).wait()
    plsc.subcore_barrier()
    pltpu.sync_copy(accum.at[pl.ds(s * STRIPE, STRIPE)],
                    out_hbm.at[c, pl.ds(s * STRIPE, STRIPE)])


@functools.cache
def _build_sc_kernels():
    mesh = plsc.VectorSubcoreMesh(core_axis_name="c", subcore_axis_name="s",
                                  num_cores=NC, num_subcores=NS)
    gather = pl.kernel(
        _sc_gather_body,
        out_type=jax.ShapeDtypeStruct((ROWS, EC, H), jnp.float32),
        mesh=mesh,
        scratch_types=[
            pltpu.VMEM((CHW, EC), jnp.int32),
            pltpu.VMEM((CHW, EC, H), jnp.float32),
            pltpu.SemaphoreType.DMA,
        ],
        compiler_params=pltpu.CompilerParams(use_tc_tiling_on_sc=False),
    )
    scatter = pl.kernel(
        _sc_scatter_body,
        out_type=jax.ShapeDtypeStruct((NC, NPAD, H), jnp.float32),
        mesh=mesh,
        scratch_types=[
            pltpu.VMEM((CHW, EC), jnp.int32),
            pltpu.VMEM((CHW, EC, H), jnp.float32),
            pltpu.VMEM_SHARED((NPAD, H), jnp.float32),
            pltpu.SemaphoreType.DMA,
        ],
        compiler_params=pltpu.CompilerParams(use_tc_tiling_on_sc=False),
    )
    return gather, scatter


def kernel(x, edge_index, edge_attr, W_proj, b_proj, W_edge, b_edge, b_nn,
           gru_W_ih, gru_W_hh, gru_b_ih, gru_b_hh,
           lstm_W_ih, lstm_W_hh, lstm_b_ih, lstm_b_hh,
           W_sp, b_sp, prelu_a):
    f32 = jnp.float32
    src3 = edge_index[0].reshape(ROWS, EC)
    dst3 = edge_index[1].reshape(ROWS, EC)

    eye = jnp.eye(H, dtype=f32)
    eye8 = jnp.eye(8, dtype=f32)
    r_mat = jnp.kron(eye8, jnp.kron(eye, jnp.ones((1, H), f32))).astype(jnp.bfloat16)
    t_mat = jnp.kron(eye8, jnp.kron(jnp.ones((1, H), f32), eye)).astype(jnp.bfloat16)
    w_mat = jnp.kron(eye8, W_edge.reshape(EDGE_IN * H, H)).astype(jnp.bfloat16)
    b3 = jnp.kron(eye8, b_edge.reshape(H, H))
    zeros_pad = jnp.zeros((NPAD, H), f32)

    bp = b_proj.reshape(1, H)
    bnn = b_nn.reshape(1, H)
    wih_t = gru_W_ih.T
    whh_t = gru_W_hh.T
    bih = gru_b_ih.reshape(1, 3 * H)
    bhh = gru_b_hh.reshape(1, 3 * H)
    lwih_t = lstm_W_ih.T
    lwhh_t = lstm_W_hh.T
    lbih = lstm_b_ih.reshape(1, 8 * H)
    lbhh = lstm_b_hh.reshape(1, 8 * H)
    bsp = b_sp.reshape(1, READOUT)
    a2 = prelu_a.reshape(1, 1)

    h0 = pl.pallas_call(
        _proj_body,
        out_shape=jax.ShapeDtypeStruct((N, H), f32),
    )(x, W_proj, bp)

    EP = EH // 8
    BLK = 2000
    msg_call = pl.pallas_call(
        _msg_body,
        grid=(EP // BLK,),
        in_specs=[
            pl.BlockSpec((BLK, 128), lambda i: (i, 0)),
            pl.BlockSpec((BLK, 128), lambda i: (i, 0)),
            pl.BlockSpec((128, 2048), lambda i: (0, 0)),
            pl.BlockSpec((128, 2048), lambda i: (0, 0)),
            pl.BlockSpec((2048, 128), lambda i: (0, 0)),
            pl.BlockSpec((128, 128), lambda i: (0, 0)),
        ],
        out_specs=pl.BlockSpec((BLK, 128), lambda i: (i, 0)),
        out_shape=jax.ShapeDtypeStruct((EP, 128), f32),
    )
    ea_p = edge_attr.reshape(EP, 128)

    gru_call = pl.pallas_call(
        _gru_body,
        out_shape=jax.ShapeDtypeStruct((NPR, 128), f32),
    )

    sc_gather, sc_scatter = _build_sc_kernels()
    h0_p = h0.reshape(NPR, 128)
    hidden_p = h0_p
    for step in range(STEPS):
        hs3 = sc_gather(hidden_p.reshape(N, H), src3)
        hs_p = hs3.reshape(EP, 128)
        msg_p = msg_call(hs_p, ea_p, r_mat, t_mat, w_mat, b3)
        msg3 = msg_p.reshape(ROWS, EC, H)
        neigh_parts = sc_scatter(msg3, dst3, zeros_pad)
        np_p = neigh_parts.reshape(NC, NPAD // 8, 128)
        if step < STEPS - 1:
            hidden_p = gru_call(np_p, hidden_p, wih_t, whh_t,
                                bih, bhh, bnn)
        else:
            out = pl.pallas_call(
                _tail_body,
                out_shape=jax.ShapeDtypeStruct((1, READOUT), f32),
            )(np_p, hidden_p, wih_t, whh_t, bih, bhh, bnn,
              h0_p, lwih_t, lwhh_t, lbih, lbhh, W_sp, bsp, a2)
    return out
